# baseline (speedup 1.0000x reference)
import functools

import jax
import jax.numpy as jnp
from jax import lax
from jax.experimental import pallas as pl
from jax.experimental.pallas import tpu as pltpu
from jax.experimental.pallas import tpu_sc as plsc

_ITEM_NUM = 1000000
_LIST_LEN = 20
_OFF = 2 * _LIST_LEN
_TOTAL = _ITEM_NUM * _LIST_LEN
_CHUNK = 25000


def kernel(x, item_list):
    info = plsc.get_sparse_core_info()
    nc, ns = info.num_cores, info.num_subcores
    nw = nc * ns
    span = _TOTAL // nw
    n_chunks = span // _CHUNK

    flat_in = item_list.reshape(-1)

    @functools.partial(
        pl.kernel,
        mesh=plsc.VectorSubcoreMesh(core_axis_name="c", subcore_axis_name="s"),
        out_type=jax.ShapeDtypeStruct((_TOTAL,), jnp.float32),
        scratch_types=[
            pltpu.VMEM((_CHUNK,), jnp.float32),
            pltpu.VMEM((_CHUNK,), jnp.float32),
            pltpu.SemaphoreType.DMA,
            pltpu.SemaphoreType.DMA,
            pltpu.SemaphoreType.DMA,
            pltpu.SemaphoreType.DMA,
        ],
    )
    def copy_kernel(in_hbm, out_hbm, buf0, buf1, is0, is1, os0, os1):
        wid = lax.axis_index("s") * nc + lax.axis_index("c")
        base_out = wid * span
        base_in = base_out + _OFF
        bufs, isems, osems = (buf0, buf1), (is0, is1), (os0, os1)
        in_h, out_h = {}, {}
        in_h[0] = pltpu.async_copy(
            in_hbm.at[pl.ds(base_in, _CHUNK)], bufs[0], isems[0])
        for i in range(n_chunks):
            b = i % 2
            if i + 1 < n_chunks:
                if i >= 1:
                    out_h[i - 1].wait()
                in_h[i + 1] = pltpu.async_copy(
                    in_hbm.at[pl.ds(base_in + (i + 1) * _CHUNK, _CHUNK)],
                    bufs[1 - b], isems[1 - b])
            in_h[i].wait()
            out_h[i] = pltpu.async_copy(
                bufs[b], out_hbm.at[pl.ds(base_out + i * _CHUNK, _CHUNK)],
                osems[b])
        out_h[n_chunks - 1].wait()
        if n_chunks >= 2:
            out_h[n_chunks - 2].wait()

    return copy_kernel(flat_in).reshape(_ITEM_NUM, _LIST_LEN)
